# Initial kernel scaffold; baseline (speedup 1.0000x reference)
import jax
import jax.numpy as jnp
from jax.experimental import pallas as pl


def kernel(x_nhwc, w1f, bias1, w2f, bias2):
    raise NotImplementedError("write your pallas kernel here")


# speedup vs baseline: 2.5297x; 2.5297x over previous
import jax
import jax.numpy as jnp
from jax.experimental import pallas as pl
from jax.experimental.pallas import tpu as pltpu

_LANE = 128


def _shift_cols_right(a):
    zero = jnp.zeros_like(a[:, :1, :])
    return jnp.concatenate([zero, a[:, :-1, :]], axis=1)


def _shift_cols_left(a):
    zero = jnp.zeros_like(a[:, :1, :])
    return jnp.concatenate([a[:, 1:, :], zero], axis=1)


def _rb_kernel(x_ref, w1_ref, w2_ref, b1_ref, b2_ref, out_ref,
               xs_ref, hs_ref):
    H, W, C = x_ref.shape
    M = H * W

    zrow = jnp.zeros((1, W, 3 * C), jnp.bfloat16)
    xs_ref[0:1] = zrow
    xs_ref[H + 1:H + 2] = zrow
    hs_ref[0:1] = zrow
    hs_ref[H + 1:H + 2] = zrow

    xb = x_ref[...].astype(jnp.bfloat16)
    xs_ref[1:H + 1, :, 0:C] = _shift_cols_right(xb)
    xs_ref[1:H + 1, :, C:2 * C] = xb
    xs_ref[1:H + 1, :, 2 * C:3 * C] = _shift_cols_left(xb)

    p1 = jnp.dot(xs_ref[...].reshape((H + 2) * W, 3 * C), w1_ref[...],
                 preferred_element_type=jnp.float32)
    a1 = (p1[0:M, 0:C] + p1[W:M + W, C:2 * C]
          + p1[2 * W:M + 2 * W, 2 * C:3 * C] + b1_ref[...])
    h = jnp.maximum(a1, 0.0).reshape(H, W, C).astype(jnp.bfloat16)
    hs_ref[1:H + 1, :, 0:C] = _shift_cols_right(h)
    hs_ref[1:H + 1, :, C:2 * C] = h
    hs_ref[1:H + 1, :, 2 * C:3 * C] = _shift_cols_left(h)

    p2 = jnp.dot(hs_ref[...].reshape((H + 2) * W, 3 * C), w2_ref[...],
                 preferred_element_type=jnp.float32)
    a2 = (p2[0:M, 0:C] + p2[W:M + W, C:2 * C]
          + p2[2 * W:M + 2 * W, 2 * C:3 * C] + b2_ref[...]
          + x_ref[...].astype(jnp.float32).reshape(M, C))
    out_ref[...] = jnp.maximum(a2, 0.0).reshape(H, W, C).astype(out_ref.dtype)


def _pack_w(w_hwio):
    kh, kw, cin, cout = w_hwio.shape
    return (w_hwio.transpose(1, 2, 0, 3)
            .reshape(kw * cin, kh * cout).astype(jnp.bfloat16))


def kernel(x_nhwc, w1f, bias1, w2f, bias2):
    N, H, W, C = x_nhwc.shape
    assert C % _LANE == 0 and W % 8 == 0, (N, H, W, C)

    w1c = _pack_w(w1f)
    w2c = _pack_w(w2f)
    b1 = bias1.astype(jnp.float32).reshape(1, C)
    b2 = bias2.astype(jnp.float32).reshape(1, C)

    def const_spec(shape):
        return pl.BlockSpec(shape, lambda n: (0, 0),
                            pipeline_mode=pl.Buffered(1))

    return pl.pallas_call(
        _rb_kernel,
        out_shape=jax.ShapeDtypeStruct((N, H, W, C), x_nhwc.dtype),
        grid=(N,),
        in_specs=[
            pl.BlockSpec((None, H, W, C), lambda n: (n, 0, 0, 0)),
            const_spec((3 * C, 3 * C)),
            const_spec((3 * C, 3 * C)),
            const_spec((1, C)),
            const_spec((1, C)),
        ],
        out_specs=pl.BlockSpec((None, H, W, C), lambda n: (n, 0, 0, 0)),
        scratch_shapes=[pltpu.VMEM((H + 2, W, 3 * C), jnp.bfloat16)
                        for _ in range(2)],
        compiler_params=pltpu.CompilerParams(
            dimension_semantics=("parallel",)),
    )(x_nhwc, w1c, w2c, b1, b2)


# speedup vs baseline: 2.5433x; 1.0053x over previous
import jax
import jax.numpy as jnp
from jax.experimental import pallas as pl
from jax.experimental.pallas import tpu as pltpu

_LANE = 128
_IMGS = 2


def _shift_cols_right(a):
    zero = jnp.zeros_like(a[:, :1, :])
    return jnp.concatenate([zero, a[:, :-1, :]], axis=1)


def _shift_cols_left(a):
    zero = jnp.zeros_like(a[:, :1, :])
    return jnp.concatenate([a[:, 1:, :], zero], axis=1)


def _rb_kernel(x_ref, w1_ref, w2_ref, b1_ref, b2_ref, out_ref,
               xs_ref, hs_ref):
    _, H, W, C = x_ref.shape
    M = H * W

    zrow = jnp.zeros((1, W, 3 * C), jnp.bfloat16)
    for i in range(_IMGS):
        xs_ref[i, 0:1] = zrow
        xs_ref[i, H + 1:H + 2] = zrow
        hs_ref[i, 0:1] = zrow
        hs_ref[i, H + 1:H + 2] = zrow

    w1 = w1_ref[...]
    w2 = w2_ref[...]
    b1 = b1_ref[...]
    b2 = b2_ref[...]

    for i in range(_IMGS):
        xb = x_ref[i].astype(jnp.bfloat16)
        xs_ref[i, 1:H + 1, :, 0:C] = _shift_cols_right(xb)
        xs_ref[i, 1:H + 1, :, C:2 * C] = xb
        xs_ref[i, 1:H + 1, :, 2 * C:3 * C] = _shift_cols_left(xb)

    for i in range(_IMGS):
        p1 = jnp.dot(xs_ref[i].reshape((H + 2) * W, 3 * C), w1,
                     preferred_element_type=jnp.float32)
        a1 = (p1[0:M, 0:C] + p1[W:M + W, C:2 * C]
              + p1[2 * W:M + 2 * W, 2 * C:3 * C] + b1)
        h = jnp.maximum(a1, 0.0).reshape(H, W, C).astype(jnp.bfloat16)
        hs_ref[i, 1:H + 1, :, 0:C] = _shift_cols_right(h)
        hs_ref[i, 1:H + 1, :, C:2 * C] = h
        hs_ref[i, 1:H + 1, :, 2 * C:3 * C] = _shift_cols_left(h)

    for i in range(_IMGS):
        p2 = jnp.dot(hs_ref[i].reshape((H + 2) * W, 3 * C), w2,
                     preferred_element_type=jnp.float32)
        a2 = (p2[0:M, 0:C] + p2[W:M + W, C:2 * C]
              + p2[2 * W:M + 2 * W, 2 * C:3 * C] + b2
              + x_ref[i].astype(jnp.float32).reshape(M, C))
        out_ref[i] = jnp.maximum(a2, 0.0).reshape(H, W, C).astype(
            out_ref.dtype)


def _pack_w(w_hwio):
    kh, kw, cin, cout = w_hwio.shape
    return (w_hwio.transpose(1, 2, 0, 3)
            .reshape(kw * cin, kh * cout).astype(jnp.bfloat16))


def kernel(x_nhwc, w1f, bias1, w2f, bias2):
    N, H, W, C = x_nhwc.shape
    assert C % _LANE == 0 and W % 8 == 0 and N % _IMGS == 0, (N, H, W, C)

    w1c = _pack_w(w1f)
    w2c = _pack_w(w2f)
    b1 = bias1.astype(jnp.float32).reshape(1, C)
    b2 = bias2.astype(jnp.float32).reshape(1, C)

    def const_spec(shape):
        return pl.BlockSpec(shape, lambda n: (0, 0),
                            pipeline_mode=pl.Buffered(1))

    return pl.pallas_call(
        _rb_kernel,
        out_shape=jax.ShapeDtypeStruct((N, H, W, C), x_nhwc.dtype),
        grid=(N // _IMGS,),
        in_specs=[
            pl.BlockSpec((_IMGS, H, W, C), lambda n: (n, 0, 0, 0)),
            const_spec((3 * C, 3 * C)),
            const_spec((3 * C, 3 * C)),
            const_spec((1, C)),
            const_spec((1, C)),
        ],
        out_specs=pl.BlockSpec((_IMGS, H, W, C), lambda n: (n, 0, 0, 0)),
        scratch_shapes=[pltpu.VMEM((_IMGS, H + 2, W, 3 * C), jnp.bfloat16)
                        for _ in range(2)],
        compiler_params=pltpu.CompilerParams(
            dimension_semantics=("parallel",)),
    )(x_nhwc, w1c, w2c, b1, b2)
